# baseline (device time: 588980 ns/iter reference)
import jax
import jax.numpy as jnp
from jax import lax
from jax.experimental import pallas as pl
from jax.experimental.pallas import tpu as pltpu

N = 32
ROWS = 128
B = 4096
D = 128
TILE = 1024


def kernel(x, Win0, Wout0, Win1, Wout1, Win2, Wout2):
    def body(x_ref, win0, wout0, win1, wout1, win2, wout2,
             out_ref, xfull, p_ref, comm, send_sems, recv_sems, bar_sem):
        my = lax.axis_index("i")
        left = (my - 1) % N
        right = (my + 1) % N

        def barrier():
            for nbr in (left, right):
                pl.semaphore_signal(
                    bar_sem, inc=1, device_id=(nbr,),
                    device_id_type=pl.DeviceIdType.MESH)
            pl.semaphore_wait(bar_sem, 2)

        def hop(h, src_ref):
            rdma = pltpu.make_async_remote_copy(
                src_ref=src_ref,
                dst_ref=comm.at[h],
                send_sem=send_sems.at[h],
                recv_sem=recv_sems.at[h],
                device_id=(right,),
                device_id_type=pl.DeviceIdType.MESH,
            )
            rdma.start()
            rdma.wait()

        def chunk(ref, c):
            return ref.at[pl.ds(c * ROWS, ROWS), :]

        xfull[pl.ds(my * ROWS, ROWS), :] = x_ref[:, :]
        for h in range(N - 1):
            hop(h, x_ref if h == 0 else comm.at[h - 1])
            origin = (my - h - 1) % N
            xfull[pl.ds(origin * ROWS, ROWS), :] = comm[h, :, :]
        barrier()

        layers = [(win0, wout0, xfull), (win1, wout1, xfull),
                  (win2, wout2, out_ref)]
        for li, (win, wout, dst) in enumerate(layers):
            for t in range(B // TILE):
                rs = pl.ds(t * TILE, TILE)
                hblk = jnp.maximum(
                    jnp.dot(xfull[rs, :], win[:, :],
                            preferred_element_type=jnp.float32), 0.0)
                p_ref[rs, :] = jnp.dot(hblk, wout[:, :],
                                       preferred_element_type=jnp.float32)

            for h in range(N - 1):
                hop(h, chunk(p_ref, (my - h) % N))
                dyn = pl.ds(((my - h - 1) % N) * ROWS, ROWS)
                p_ref[dyn, :] = p_ref[dyn, :] + comm[h, :, :]
            barrier()

            own = (my + 1) % N
            dst[pl.ds(own * ROWS, ROWS), :] = p_ref[pl.ds(own * ROWS, ROWS), :]
            for h in range(N - 1):
                hop(h, chunk(p_ref, own) if h == 0 else comm.at[h - 1])
                cid = (my - h) % N
                dst[pl.ds(cid * ROWS, ROWS), :] = comm[h, :, :]
            if li < 2:
                barrier()

    return pl.pallas_call(
        body,
        out_shape=jax.ShapeDtypeStruct((B, D), jnp.float32),
        in_specs=[pl.BlockSpec(memory_space=pltpu.VMEM)] * 7,
        out_specs=pl.BlockSpec(memory_space=pltpu.VMEM),
        scratch_shapes=[
            pltpu.VMEM((B, D), jnp.float32),
            pltpu.VMEM((B, D), jnp.float32),
            pltpu.VMEM((N - 1, ROWS, D), jnp.float32),
            pltpu.SemaphoreType.DMA((N - 1,)),
            pltpu.SemaphoreType.DMA((N - 1,)),
            pltpu.SemaphoreType.REGULAR,
        ],
    )(x, Win0, Wout0, Win1, Wout1, Win2, Wout2)
